# baseline (speedup 1.0000x reference)
import functools

import jax
import jax.numpy as jnp
from jax import lax
from jax.experimental import pallas as pl
from jax.experimental.pallas import tpu as pltpu, tpu_sc as plsc

BATCH = 16384
DIM = 64
NROWS = 1000000
OUT_W = 128
CAP = 128

_info = plsc.get_sparse_core_info()
_NC, _NS = _info.num_cores, _info.num_subcores
_NW = _NC * _NS
_RANGE = 31232
_CHUNK = 512
_NCH = 62
_IDS_SUB = 4096

_TAIL_B = NROWS - 64

_mesh = plsc.VectorSubcoreMesh(core_axis_name="c", subcore_axis_name="s")


@functools.partial(
    pl.kernel,
    mesh=_mesh,
    out_type=jax.ShapeDtypeStruct((BATCH, OUT_W), jnp.float32),
    scratch_types=[
        pltpu.VMEM((_IDS_SUB,), jnp.int32),
        pltpu.VMEM((BATCH,), jnp.int32),
        pltpu.VMEM((BATCH,), jnp.int32),
        pltpu.VMEM((2, DIM, _CHUNK), jnp.float32),
        pltpu.VMEM((DIM, DIM), jnp.float32),
        pltpu.VMEM((CAP,), jnp.int32),
        pltpu.VMEM((CAP,), jnp.int32),
        pltpu.VMEM((CAP, OUT_W), jnp.float32),
        pltpu.SemaphoreType.DMA,
        pltpu.SemaphoreType.DMA,
        pltpu.SemaphoreType.DMA,
    ],
    compiler_params=pltpu.CompilerParams(needs_layout_passes=False),
)
def _gather_sc(ids_hbm, table_hbm, tail_hbm, out_hbm, idsb_v, cid_v, cpos_v,
               chunk_v, tail_v, lcol_v, sidx_v, stage_v, sem_a, sem_b, sem_s):
    wid = lax.axis_index("s") * _NC + lax.axis_index("c")
    lo = wid * _RANGE
    hi = jnp.where(wid == _NW - 1, NROWS, lo + _RANGE)
    iota = lax.iota(jnp.int32, 16)
    neg1 = jnp.full((16,), -1, jnp.int32)

    def issue(c, buf, sem):
        c0 = pl.multiple_of(lo + c * _CHUNK, 128)
        pltpu.async_copy(
            table_hbm.at[:, pl.ds(c0, _CHUNK)], chunk_v.at[buf], sem
        )

    def wait_chunk(sem):
        pltpu.make_async_copy(
            table_hbm.at[:, pl.ds(0, _CHUNK)], chunk_v.at[0], sem
        ).wait()

    issue(0, 0, sem_a)
    issue(1, 1, sem_b)

    for q in range(CAP // 16):
        sidx_v[pl.ds(q * 16, 16)] = neg1

    def sub_batch(b, n):
        pltpu.sync_copy(ids_hbm.at[pl.ds(b * _IDS_SUB, _IDS_SUB)], idsb_v)

        def grp(t, n):
            v = idsb_v[pl.ds(t * 16, 16)]
            m = (v >= lo) & (v < hi)
            cum = plsc.cumsum(jnp.where(m, 1, 0))
            pos = cum + (n - 1)
            plsc.store_scatter(cid_v, [pos], v, mask=m)
            plsc.store_scatter(
                cpos_v, [pos], iota + (b * _IDS_SUB + t * 16), mask=m
            )
            return n + jnp.sum(jnp.where(m, 1, 0))

        return lax.fori_loop(0, _IDS_SUB // 16, grp, n)

    n = lax.fori_loop(0, BATCH // _IDS_SUB, sub_batch, jnp.int32(0))
    ngrp = (n + 15) // 16

    def drain_scatter():
        pltpu.make_async_copy(
            out_hbm.at[pl.ds(0, CAP), :], stage_v, sem_s
        ).wait()

    def scan_window(c0, span, base):
        def scan(g, kc):
            v = cid_v[pl.ds(g * 16, 16)]
            m = (iota < (n - g * 16)) & (v >= c0) & (v < c0 + span)
            cnt = jnp.sum(jnp.where(m, 1, 0))

            @pl.when(cnt > 0)
            def _():
                p = cpos_v[pl.ds(g * 16, 16)]
                rank = kc + plsc.cumsum(jnp.where(m, 1, 0)) - 1
                sel = m & (rank >= base) & (rank < base + CAP)
                plsc.store_scatter(lcol_v, [rank - base], v - c0, mask=sel)
                plsc.store_scatter(sidx_v, [rank - base], p, mask=sel)

            return kc + cnt

        return lax.fori_loop(0, ngrp, scan, jnp.int32(0))

    def extract_fire(gather_fn, nvalid, outst):
        def extract(e, _):
            em = iota < (nvalid - e * 16)
            lvs = lcol_v[pl.ds(e * 16, 16)]
            for cc in range(DIM):
                col = jnp.full((16,), cc, jnp.int32)
                vals = gather_fn(col, lvs, em)
                plsc.store_scatter(
                    stage_v, [iota + e * 16, col], vals, mask=em
                )
            return ()

        lax.fori_loop(0, (nvalid + 15) // 16, extract, ())

        @pl.when(nvalid > 0)
        def _():
            pltpu.async_copy(
                stage_v,
                out_hbm.at[plsc.Indices(sidx_v, ignored_value=-1)],
                sem_s,
            )

        return jnp.where(nvalid > 0, jnp.int32(1), outst)

    def chunk_gather(buf):
        def g(col, lvs, em):
            b = jnp.full((16,), buf, jnp.int32)
            return plsc.load_gather(chunk_v, [b, col, lvs], mask=em)

        return g

    def overflow_rounds(gather_fn, c0, span, kk, outst):
        def rnd(r, outst):
            @pl.when(outst == 1)
            def _():
                drain_scatter()

            scan_window(c0, span, r * CAP)
            return extract_fire(
                gather_fn, jnp.minimum(kk - r * CAP, CAP), outst
            )

        return lax.fori_loop(1, (kk + CAP - 1) // CAP, rnd, outst)

    def half(c, buf, sem, outst):
        c0 = lo + c * _CHUNK

        @pl.when(outst == 1)
        def _():
            drain_scatter()

        outst = jnp.int32(0)
        kk = scan_window(c0, _CHUNK, 0)
        wait_chunk(sem)
        outst = extract_fire(chunk_gather(buf), jnp.minimum(kk, CAP), outst)
        outst = overflow_rounds(chunk_gather(buf), c0, _CHUNK, kk, outst)
        return outst

    def pair(i, outst):
        outst = half(i * 2, 0, sem_a, outst)

        @pl.when(i * 2 + 2 < _NCH)
        def _():
            issue(i * 2 + 2, 0, sem_a)

        outst = half(i * 2 + 1, 1, sem_b, outst)

        @pl.when(i * 2 + 3 < _NCH)
        def _():
            issue(i * 2 + 3, 1, sem_b)

        return outst

    outst = lax.fori_loop(0, _NCH // 2, pair, jnp.int32(0))
    if _NCH % 2:
        outst = half(_NCH - 1, 0, sem_a, outst)

    @pl.when(wid == _NW - 1)
    def _():
        pltpu.sync_copy(tail_hbm, tail_v)

        def g(col, lvs, em):
            return plsc.load_gather(tail_v, [col, lvs], mask=em)

        @pl.when(outst == 1)
        def _():
            drain_scatter()

        kk = scan_window(jnp.int32(_TAIL_B), NROWS - _TAIL_B, 0)
        outst2 = extract_fire(g, jnp.minimum(kk, CAP), jnp.int32(0))
        outst2 = overflow_rounds(
            g, jnp.int32(_TAIL_B), NROWS - _TAIL_B, kk, outst2
        )

        @pl.when(outst2 == 1)
        def _():
            drain_scatter()

    @pl.when((wid != _NW - 1) & (outst == 1))
    def _():
        drain_scatter()


def kernel(ids, table):
    tail = table[_TAIL_B:, :].T
    out_wide = _gather_sc(ids.astype(jnp.int32), table.T, tail)
    return out_wide[:, :DIM]


# speedup vs baseline: 1.1844x; 1.1844x over previous
import functools

import jax
import jax.numpy as jnp
from jax import lax
from jax.experimental import pallas as pl
from jax.experimental.pallas import tpu as pltpu, tpu_sc as plsc

BATCH = 16384
DIM = 64
NROWS = 1000000
OUT_W = 128
CAP = 128

_info = plsc.get_sparse_core_info()
_NC, _NS = _info.num_cores, _info.num_subcores
_NW = _NC * _NS
_RANGE = 31232
_CHUNK = 512
_NCH = 62
_IDS_SUB = 4096

_TAIL_B = NROWS - 64

_mesh = plsc.VectorSubcoreMesh(core_axis_name="c", subcore_axis_name="s")


@functools.partial(
    pl.kernel,
    mesh=_mesh,
    out_type=jax.ShapeDtypeStruct((BATCH, OUT_W), jnp.float32),
    scratch_types=[
        pltpu.VMEM((_IDS_SUB,), jnp.int32),
        pltpu.VMEM((BATCH,), jnp.int32),
        pltpu.VMEM((BATCH,), jnp.int32),
        pltpu.VMEM((2, DIM, _CHUNK), jnp.float32),
        pltpu.VMEM((DIM, DIM), jnp.float32),
        pltpu.VMEM((CAP,), jnp.int32),
        pltpu.VMEM((CAP,), jnp.int32),
        pltpu.VMEM((CAP, OUT_W), jnp.float32),
        pltpu.SemaphoreType.DMA,
        pltpu.SemaphoreType.DMA,
        pltpu.SemaphoreType.DMA,
    ],
    compiler_params=pltpu.CompilerParams(needs_layout_passes=False),
)
def _gather_sc(ids_hbm, table_hbm, tail_hbm, out_hbm, idsb_v, cid_v, cpos_v,
               chunk_v, tail_v, lcol_v, sidx_v, stage_v, sem_a, sem_b, sem_s):
    wid = lax.axis_index("s") * _NC + lax.axis_index("c")
    lo = wid * _RANGE
    hi = jnp.where(wid == _NW - 1, NROWS, lo + _RANGE)
    iota = lax.iota(jnp.int32, 16)
    neg1 = jnp.full((16,), -1, jnp.int32)

    def issue(c, buf, sem):
        c0 = pl.multiple_of(lo + c * _CHUNK, 128)
        pltpu.async_copy(
            table_hbm.at[:, pl.ds(c0, _CHUNK)], chunk_v.at[buf], sem
        )

    def wait_chunk(sem):
        pltpu.make_async_copy(
            table_hbm.at[:, pl.ds(0, _CHUNK)], chunk_v.at[0], sem
        ).wait()

    issue(0, 0, sem_a)
    issue(1, 1, sem_b)

    for q in range(CAP // 16):
        sidx_v[pl.ds(q * 16, 16)] = neg1

    def sub_batch(b, n):
        pltpu.sync_copy(ids_hbm.at[pl.ds(b * _IDS_SUB, _IDS_SUB)], idsb_v)

        def grp(t, n):
            v = idsb_v[pl.ds(t * 16, 16)]
            m = (v >= lo) & (v < hi)
            cum = plsc.cumsum(jnp.where(m, 1, 0))
            pos = cum + (n - 1)
            plsc.store_scatter(cid_v, [pos], v, mask=m)
            plsc.store_scatter(
                cpos_v, [pos], iota + (b * _IDS_SUB + t * 16), mask=m
            )
            return n + jnp.sum(jnp.where(m, 1, 0))

        return lax.fori_loop(0, _IDS_SUB // 16, grp, n)

    n = lax.fori_loop(0, BATCH // _IDS_SUB, sub_batch, jnp.int32(0))
    ngrp = (n + 15) // 16

    def drain_scatter():
        pltpu.make_async_copy(
            out_hbm.at[pl.ds(0, CAP), :], stage_v, sem_s
        ).wait()

    def scan_window(c0, span, base):
        def scan(g, kc):
            v = cid_v[pl.ds(g * 16, 16)]
            p = cpos_v[pl.ds(g * 16, 16)]
            m = (iota < (n - g * 16)) & (v >= c0) & (v < c0 + span)
            rank = kc + plsc.cumsum(jnp.where(m, 1, 0)) - 1
            sel = m & (rank >= base) & (rank < base + CAP)
            plsc.store_scatter(lcol_v, [rank - base], v - c0, mask=sel)
            plsc.store_scatter(sidx_v, [rank - base], p, mask=sel)
            return kc + jnp.sum(jnp.where(m, 1, 0))

        return lax.fori_loop(0, ngrp, scan, jnp.int32(0))

    def extract_fire(gather_fn, nvalid, outst):
        def extract(e, _):
            em = iota < (nvalid - e * 16)
            lvs = lcol_v[pl.ds(e * 16, 16)]
            for cc in range(DIM):
                col = jnp.full((16,), cc, jnp.int32)
                vals = gather_fn(col, lvs, em)
                plsc.store_scatter(
                    stage_v, [iota + e * 16, col], vals, mask=em
                )
            return ()

        lax.fori_loop(0, (nvalid + 15) // 16, extract, ())

        @pl.when(nvalid > 0)
        def _():
            pltpu.async_copy(
                stage_v,
                out_hbm.at[plsc.Indices(sidx_v, ignored_value=-1)],
                sem_s,
            )

        return jnp.where(nvalid > 0, jnp.int32(1), outst)

    def chunk_gather(buf):
        def g(col, lvs, em):
            b = jnp.full((16,), buf, jnp.int32)
            return plsc.load_gather(chunk_v, [b, col, lvs], mask=em)

        return g

    def overflow_rounds(gather_fn, c0, span, kk, outst):
        def rnd(r, outst):
            @pl.when(outst == 1)
            def _():
                drain_scatter()

            scan_window(c0, span, r * CAP)
            return extract_fire(
                gather_fn, jnp.minimum(kk - r * CAP, CAP), outst
            )

        return lax.fori_loop(1, (kk + CAP - 1) // CAP, rnd, outst)

    def half(c, buf, sem, outst):
        c0 = lo + c * _CHUNK

        @pl.when(outst == 1)
        def _():
            drain_scatter()

        outst = jnp.int32(0)
        kk = scan_window(c0, _CHUNK, 0)
        wait_chunk(sem)
        outst = extract_fire(chunk_gather(buf), jnp.minimum(kk, CAP), outst)
        outst = overflow_rounds(chunk_gather(buf), c0, _CHUNK, kk, outst)
        return outst

    def pair(i, outst):
        outst = half(i * 2, 0, sem_a, outst)

        @pl.when(i * 2 + 2 < _NCH)
        def _():
            issue(i * 2 + 2, 0, sem_a)

        outst = half(i * 2 + 1, 1, sem_b, outst)

        @pl.when(i * 2 + 3 < _NCH)
        def _():
            issue(i * 2 + 3, 1, sem_b)

        return outst

    outst = lax.fori_loop(0, _NCH // 2, pair, jnp.int32(0))
    if _NCH % 2:
        outst = half(_NCH - 1, 0, sem_a, outst)

    @pl.when(wid == _NW - 1)
    def _():
        pltpu.sync_copy(tail_hbm, tail_v)

        def g(col, lvs, em):
            return plsc.load_gather(tail_v, [col, lvs], mask=em)

        @pl.when(outst == 1)
        def _():
            drain_scatter()

        kk = scan_window(jnp.int32(_TAIL_B), NROWS - _TAIL_B, 0)
        outst2 = extract_fire(g, jnp.minimum(kk, CAP), jnp.int32(0))
        outst2 = overflow_rounds(
            g, jnp.int32(_TAIL_B), NROWS - _TAIL_B, kk, outst2
        )

        @pl.when(outst2 == 1)
        def _():
            drain_scatter()

    @pl.when((wid != _NW - 1) & (outst == 1))
    def _():
        drain_scatter()


def kernel(ids, table):
    tail = table[_TAIL_B:, :].T
    out_wide = _gather_sc(ids.astype(jnp.int32), table.T, tail)
    return out_wide[:, :DIM]
